# baseline (speedup 1.0000x reference)
import functools

import jax
import jax.numpy as jnp
from jax import lax
from jax.experimental import pallas as pl
from jax.experimental.pallas import tpu as pltpu
from jax.experimental.pallas import tpu_sc as plsc

N_USERS = 1000000
N_ITEMS = 100000
K = 32
B = 16384
MU = 3.5

_INFO = plsc.get_sparse_core_info()
NC, NS, L = _INFO.num_cores, _INFO.num_subcores, _INFO.num_lanes
NW = NC * NS
BPW = B // NW
GROUPS = BPW // L


def _lfm_kernel(uidx_hbm, iidx_hbm, pt_hbm, qt_hbm, bu_hbm, bi_hbm, out_hbm,
                uidx_v, iidx_v, pidx_v, qidx_v, p_v, q_v, bu_v, bi_v, o_v, sem):
    wid = lax.axis_index("s") * NC + lax.axis_index("c")
    base = wid * BPW

    pltpu.sync_copy(uidx_hbm.at[pl.ds(base, BPW)], uidx_v)
    pltpu.sync_copy(iidx_hbm.at[pl.ds(base, BPW)], iidx_v)

    def mkidx(j, carry):
        u16 = uidx_v[pl.ds(j * L, L)]
        i16 = iidx_v[pl.ds(j * L, L)]
        for k in range(K):
            pidx_v[pl.ds(k * BPW + j * L, L)] = u16 + k * N_USERS
            qidx_v[pl.ds(k * BPW + j * L, L)] = i16 + k * N_ITEMS
        return carry

    lax.fori_loop(0, BPW // L, mkidx, 0)

    cps = [
        pltpu.async_copy(pt_hbm.at[pidx_v], p_v, sem),
        pltpu.async_copy(qt_hbm.at[qidx_v], q_v, sem),
        pltpu.async_copy(bu_hbm.at[uidx_v], bu_v, sem),
        pltpu.async_copy(bi_hbm.at[iidx_v], bi_v, sem),
    ]
    for cp in cps:
        cp.wait()

    def group(g, carry):
        acc = MU + bu_v[pl.ds(g * L, L)] + bi_v[pl.ds(g * L, L)]
        for k in range(K):
            off = k * BPW + g * L
            acc = acc + p_v[pl.ds(off, L)] * q_v[pl.ds(off, L)]
        o_v[pl.ds(g * L, L)] = acc
        return carry

    lax.fori_loop(0, GROUPS, group, 0)
    pltpu.sync_copy(o_v, out_hbm.at[pl.ds(base, BPW)])


@jax.jit
def kernel(user_idx, item_idx, P, Q, b_u, b_i):
    mesh = plsc.VectorSubcoreMesh(core_axis_name="c", subcore_axis_name="s")
    run = functools.partial(
        pl.kernel,
        mesh=mesh,
        out_type=jax.ShapeDtypeStruct((B,), jnp.float32),
        scratch_types=[
            pltpu.VMEM((BPW,), jnp.int32),
            pltpu.VMEM((BPW,), jnp.int32),
            pltpu.VMEM((BPW * K,), jnp.int32),
            pltpu.VMEM((BPW * K,), jnp.int32),
            pltpu.VMEM((BPW * K,), jnp.float32),
            pltpu.VMEM((BPW * K,), jnp.float32),
            pltpu.VMEM((BPW,), jnp.float32),
            pltpu.VMEM((BPW,), jnp.float32),
            pltpu.VMEM((BPW,), jnp.float32),
            pltpu.SemaphoreType.DMA,
        ],
        compiler_params=pltpu.CompilerParams(
            needs_layout_passes=False, use_tc_tiling_on_sc=False),
    )(_lfm_kernel)
    return run(user_idx, item_idx, P.T.reshape(-1), Q.T.reshape(-1),
               b_u.reshape(-1), b_i.reshape(-1))


# speedup vs baseline: 10.3024x; 10.3024x over previous
import functools

import jax
import jax.numpy as jnp
from jax import lax
from jax.experimental import pallas as pl
from jax.experimental.pallas import tpu as pltpu
from jax.experimental.pallas import tpu_sc as plsc

N_USERS = 1000000
N_ITEMS = 100000
K = 32
B = 16384
MU = 3.5

_INFO = plsc.get_sparse_core_info()
NC, NS, L = _INFO.num_cores, _INFO.num_subcores, _INFO.num_lanes
NW = NC * NS
BPW = B // NW
CB = 128
NCHUNK = BPW // CB
CGROUPS = CB // L


def _pack_body(pt_ref, out_ref):
    x = pt_ref[...]
    c = x.shape[1]
    r = c // 4
    y = x.reshape(K, 4, r).transpose(1, 0, 2).reshape(4 * K, r)
    out_ref[...] = y.T


def _pack(table_t, chunk):
    n = table_t.shape[1]
    grid = (n + chunk - 1) // chunk
    return pl.pallas_call(
        _pack_body,
        grid=(grid,),
        in_specs=[pl.BlockSpec((K, chunk), lambda c: (0, c))],
        out_specs=pl.BlockSpec((chunk // 4, 4 * K), lambda c: (c, 0)),
        out_shape=jax.ShapeDtypeStruct((grid * chunk // 4, 4 * K),
                                       jnp.float32),
    )(table_t)


def _lfm_kernel(uidx_hbm, iidx_hbm, p_hbm, q_hbm, bu_hbm, bi_hbm, out_hbm,
                uidx_v, iidx_v, sup_v, p_v, q_v, bu_v, bi_v, o_v, sem):
    wid = lax.axis_index("s") * NC + lax.axis_index("c")
    base = wid * BPW

    pltpu.sync_copy(uidx_hbm.at[pl.ds(base, BPW)], uidx_v)
    pltpu.sync_copy(iidx_hbm.at[pl.ds(base, BPW)], iidx_v)

    lane = lax.iota(jnp.int32, L)

    def chunk(c, carry):
        c0 = c * CB

        def mkidx(j, carry):
            u16 = uidx_v[pl.ds(c0 + j * L, L)]
            i16 = iidx_v[pl.ds(c0 + j * L, L)]
            sup_v[0, pl.ds(j * L, L)] = ((u16 >> 13) << 11) + (u16 & 2047)
            sup_v[1, pl.ds(j * L, L)] = ((i16 >> 12) << 10) + (i16 & 1023)
            sup_v[2, pl.ds(j * L, L)] = u16 >> 7
            sup_v[3, pl.ds(j * L, L)] = i16 >> 7
            return carry

        lax.fori_loop(0, CB // L, mkidx, 0)

        cps = [
            pltpu.async_copy(p_hbm.at[sup_v.at[0]], p_v, sem),
            pltpu.async_copy(q_hbm.at[sup_v.at[1]], q_v, sem),
            pltpu.async_copy(bu_hbm.at[sup_v.at[2]], bu_v, sem),
            pltpu.async_copy(bi_hbm.at[sup_v.at[3]], bi_v, sem),
        ]
        for cp in cps:
            cp.wait()

        def group(g, carry):
            rows = g * L + lane
            u16 = uidx_v[pl.ds(c0 + g * L, L)]
            i16 = iidx_v[pl.ds(c0 + g * L, L)]
            acc = (MU + plsc.load_gather(bu_v, [rows, u16 & 127])
                   + plsc.load_gather(bi_v, [rows, i16 & 127]))
            usub = ((u16 >> 11) & 3) * K
            isub = ((i16 >> 10) & 3) * K
            for k in range(K):
                pk = plsc.load_gather(p_v, [rows, usub + k])
                qk = plsc.load_gather(q_v, [rows, isub + k])
                acc = acc + pk * qk
            o_v[pl.ds(c0 + g * L, L)] = acc
            return carry

        lax.fori_loop(0, CGROUPS, group, 0)
        return carry

    lax.fori_loop(0, NCHUNK, chunk, 0)
    pltpu.sync_copy(o_v, out_hbm.at[pl.ds(base, BPW)])


@jax.jit
def kernel(user_idx, item_idx, P, Q, b_u, b_i):
    mesh = plsc.VectorSubcoreMesh(core_axis_name="c", subcore_axis_name="s")
    run = functools.partial(
        pl.kernel,
        mesh=mesh,
        out_type=jax.ShapeDtypeStruct((B,), jnp.float32),
        scratch_types=[
            pltpu.VMEM((BPW,), jnp.int32),
            pltpu.VMEM((BPW,), jnp.int32),
            pltpu.VMEM((4, CB), jnp.int32),
            pltpu.VMEM((CB, 128), jnp.float32),
            pltpu.VMEM((CB, 128), jnp.float32),
            pltpu.VMEM((CB, 128), jnp.float32),
            pltpu.VMEM((CB, 128), jnp.float32),
            pltpu.VMEM((BPW,), jnp.float32),
            pltpu.SemaphoreType.DMA,
        ],
        compiler_params=pltpu.CompilerParams(
            needs_layout_passes=False, use_tc_tiling_on_sc=True),
    )(_lfm_kernel)
    p_packed = _pack(P.T, 8192)
    q_packed = _pack(Q.T, 4096)
    bu_p = jnp.pad(b_u.reshape(-1), (0, 64)).reshape(-1, 128)
    bi_p = jnp.pad(b_i.reshape(-1), (0, 96)).reshape(-1, 128)
    return run(user_idx, item_idx, p_packed, q_packed, bu_p, bi_p)


# speedup vs baseline: 12.0011x; 1.1649x over previous
import functools

import jax
import jax.numpy as jnp
from jax import lax
from jax.experimental import pallas as pl
from jax.experimental.pallas import tpu as pltpu
from jax.experimental.pallas import tpu_sc as plsc

N_USERS = 1000000
N_ITEMS = 100000
K = 32
B = 16384
MU = 3.5

_INFO = plsc.get_sparse_core_info()
NC, NS, L = _INFO.num_cores, _INFO.num_subcores, _INFO.num_lanes
NW = NC * NS
BPW = B // NW
CB = 128
NCHUNK = BPW // CB
CGROUPS = CB // L


def _pack_body(pt_ref, out_ref):
    x = pt_ref[...]
    c = x.shape[1]
    r = c // 4
    y = x.reshape(K, 4, r).transpose(1, 0, 2).reshape(4 * K, r)
    out_ref[...] = y.T


def _pack(table_t, chunk):
    n = table_t.shape[1]
    grid = (n + chunk - 1) // chunk
    return pl.pallas_call(
        _pack_body,
        grid=(grid,),
        in_specs=[pl.BlockSpec((K, chunk), lambda c: (0, c))],
        out_specs=pl.BlockSpec((chunk // 4, 4 * K), lambda c: (c, 0)),
        out_shape=jax.ShapeDtypeStruct((grid * chunk // 4, 4 * K),
                                       jnp.float32),
    )(table_t)


def _lfm_kernel(uidx_hbm, iidx_hbm, p_hbm, q_hbm, bu_hbm, bi_hbm, out_hbm,
                uidx_v, iidx_v, sup_v, p_v, q_v, bu_v, bi_v, o_v, sem):
    wid = lax.axis_index("s") * NC + lax.axis_index("c")
    base = wid * BPW

    pltpu.sync_copy(uidx_hbm.at[pl.ds(base, BPW)], uidx_v)
    pltpu.sync_copy(iidx_hbm.at[pl.ds(base, BPW)], iidx_v)

    lane = lax.iota(jnp.int32, L)

    def chunk(c, carry):
        c0 = c * CB

        def mkidx(j, carry):
            u16 = uidx_v[pl.ds(c0 + j * L, L)]
            i16 = iidx_v[pl.ds(c0 + j * L, L)]
            sup_v[0, pl.ds(j * L, L)] = ((u16 >> 14) << 12) + (u16 & 4095)
            sup_v[1, pl.ds(j * L, L)] = ((i16 >> 12) << 10) + (i16 & 1023)
            sup_v[2, pl.ds(j * L, L)] = u16 >> 7
            sup_v[3, pl.ds(j * L, L)] = i16 >> 7
            return carry

        lax.fori_loop(0, CB // L, mkidx, 0)

        cps = [
            pltpu.async_copy(p_hbm.at[sup_v.at[0]], p_v, sem),
            pltpu.async_copy(q_hbm.at[sup_v.at[1]], q_v, sem),
            pltpu.async_copy(bu_hbm.at[sup_v.at[2]], bu_v, sem),
            pltpu.async_copy(bi_hbm.at[sup_v.at[3]], bi_v, sem),
        ]
        for cp in cps:
            cp.wait()

        def group(g, carry):
            rows = g * L + lane
            u16 = uidx_v[pl.ds(c0 + g * L, L)]
            i16 = iidx_v[pl.ds(c0 + g * L, L)]
            acc = (MU + plsc.load_gather(bu_v, [rows, u16 & 127])
                   + plsc.load_gather(bi_v, [rows, i16 & 127]))
            usub = ((u16 >> 12) & 3) * K
            isub = ((i16 >> 10) & 3) * K
            for k in range(K):
                pk = plsc.load_gather(p_v, [rows, usub + k])
                qk = plsc.load_gather(q_v, [rows, isub + k])
                acc = acc + pk * qk
            o_v[pl.ds(c0 + g * L, L)] = acc
            return carry

        lax.fori_loop(0, CGROUPS, group, 0)
        return carry

    lax.fori_loop(0, NCHUNK, chunk, 0)
    pltpu.sync_copy(o_v, out_hbm.at[pl.ds(base, BPW)])


@jax.jit
def kernel(user_idx, item_idx, P, Q, b_u, b_i):
    mesh = plsc.VectorSubcoreMesh(core_axis_name="c", subcore_axis_name="s")
    run = functools.partial(
        pl.kernel,
        mesh=mesh,
        out_type=jax.ShapeDtypeStruct((B,), jnp.float32),
        scratch_types=[
            pltpu.VMEM((BPW,), jnp.int32),
            pltpu.VMEM((BPW,), jnp.int32),
            pltpu.VMEM((4, CB), jnp.int32),
            pltpu.VMEM((CB, 128), jnp.float32),
            pltpu.VMEM((CB, 128), jnp.float32),
            pltpu.VMEM((CB, 128), jnp.float32),
            pltpu.VMEM((CB, 128), jnp.float32),
            pltpu.VMEM((BPW,), jnp.float32),
            pltpu.SemaphoreType.DMA,
        ],
        compiler_params=pltpu.CompilerParams(
            needs_layout_passes=False, use_tc_tiling_on_sc=True),
    )(_lfm_kernel)
    p_packed = _pack(P.T, 16384)
    q_packed = _pack(Q.T, 4096)
    bu_p = jnp.pad(b_u.reshape(-1), (0, 64)).reshape(-1, 128)
    bi_p = jnp.pad(b_i.reshape(-1), (0, 96)).reshape(-1, 128)
    return run(user_idx, item_idx, p_packed, q_packed, bu_p, bi_p)


# speedup vs baseline: 12.9854x; 1.0820x over previous
import functools

import jax
import jax.numpy as jnp
from jax import lax
from jax.experimental import pallas as pl
from jax.experimental.pallas import tpu as pltpu
from jax.experimental.pallas import tpu_sc as plsc

N_USERS = 1000000
N_ITEMS = 100000
K = 32
B = 16384
MU = 3.5

_INFO = plsc.get_sparse_core_info()
NC, NS, L = _INFO.num_cores, _INFO.num_subcores, _INFO.num_lanes
NW = NC * NS
BPW = B // NW
CB = 128
NCHUNK = BPW // CB
CGROUPS = CB // L


def _pack_body(pt_ref, out_ref):
    x = pt_ref[...]
    c = x.shape[1]
    r = c // 4
    y = x.reshape(K, 4, r).transpose(1, 0, 2).reshape(4 * K, r)
    out_ref[...] = y.T


def _pack(table_t, chunk):
    n = table_t.shape[1]
    grid = (n + chunk - 1) // chunk
    return pl.pallas_call(
        _pack_body,
        grid=(grid,),
        in_specs=[pl.BlockSpec((K, chunk), lambda c: (0, c))],
        out_specs=pl.BlockSpec((chunk // 4, 4 * K), lambda c: (c, 0)),
        out_shape=jax.ShapeDtypeStruct((grid * chunk // 4, 4 * K),
                                       jnp.float32),
    )(table_t)


def _lfm_kernel(uidx_hbm, iidx_hbm, p_hbm, q_hbm, bu_hbm, bi_hbm, out_hbm,
                uidx_v, iidx_v, sup_v, p_v, q_v, bu_v, bi_v, o_v, sem):
    wid = lax.axis_index("s") * NC + lax.axis_index("c")
    base = wid * BPW

    pltpu.sync_copy(uidx_hbm.at[pl.ds(base, BPW)], uidx_v)
    pltpu.sync_copy(iidx_hbm.at[pl.ds(base, BPW)], iidx_v)

    lane = lax.iota(jnp.int32, L)

    def chunk(c, carry):
        c0 = c * CB

        def mkidx(j, carry):
            u16 = uidx_v[pl.ds(c0 + j * L, L)]
            i16 = iidx_v[pl.ds(c0 + j * L, L)]
            sup_v[0, pl.ds(j * L, L)] = ((u16 >> 15) << 13) + (u16 & 8191)
            sup_v[1, pl.ds(j * L, L)] = ((i16 >> 12) << 10) + (i16 & 1023)
            sup_v[2, pl.ds(j * L, L)] = u16 >> 7
            sup_v[3, pl.ds(j * L, L)] = i16 >> 7
            return carry

        lax.fori_loop(0, CB // L, mkidx, 0)

        cps = [
            pltpu.async_copy(p_hbm.at[sup_v.at[0]], p_v, sem),
            pltpu.async_copy(q_hbm.at[sup_v.at[1]], q_v, sem),
            pltpu.async_copy(bu_hbm.at[sup_v.at[2]], bu_v, sem),
            pltpu.async_copy(bi_hbm.at[sup_v.at[3]], bi_v, sem),
        ]
        for cp in cps:
            cp.wait()

        def group(g, carry):
            rows = g * L + lane
            u16 = uidx_v[pl.ds(c0 + g * L, L)]
            i16 = iidx_v[pl.ds(c0 + g * L, L)]
            acc = (MU + plsc.load_gather(bu_v, [rows, u16 & 127])
                   + plsc.load_gather(bi_v, [rows, i16 & 127]))
            usub = ((u16 >> 13) & 3) * K
            isub = ((i16 >> 10) & 3) * K
            for k in range(K):
                pk = plsc.load_gather(p_v, [rows, usub + k])
                qk = plsc.load_gather(q_v, [rows, isub + k])
                acc = acc + pk * qk
            o_v[pl.ds(c0 + g * L, L)] = acc
            return carry

        lax.fori_loop(0, CGROUPS, group, 0)
        return carry

    lax.fori_loop(0, NCHUNK, chunk, 0)
    pltpu.sync_copy(o_v, out_hbm.at[pl.ds(base, BPW)])


@jax.jit
def kernel(user_idx, item_idx, P, Q, b_u, b_i):
    mesh = plsc.VectorSubcoreMesh(core_axis_name="c", subcore_axis_name="s")
    run = functools.partial(
        pl.kernel,
        mesh=mesh,
        out_type=jax.ShapeDtypeStruct((B,), jnp.float32),
        scratch_types=[
            pltpu.VMEM((BPW,), jnp.int32),
            pltpu.VMEM((BPW,), jnp.int32),
            pltpu.VMEM((4, CB), jnp.int32),
            pltpu.VMEM((CB, 128), jnp.float32),
            pltpu.VMEM((CB, 128), jnp.float32),
            pltpu.VMEM((CB, 128), jnp.float32),
            pltpu.VMEM((CB, 128), jnp.float32),
            pltpu.VMEM((BPW,), jnp.float32),
            pltpu.SemaphoreType.DMA,
        ],
        compiler_params=pltpu.CompilerParams(
            needs_layout_passes=False, use_tc_tiling_on_sc=True),
    )(_lfm_kernel)
    p_packed = _pack(P.T, 32768)
    q_packed = _pack(Q.T, 4096)
    bu_p = jnp.pad(b_u, ((0, 64), (0, 0))).reshape(-1, 128)
    bi_p = jnp.pad(b_i, ((0, 96), (0, 0))).reshape(-1, 128)
    return run(user_idx, item_idx, p_packed, q_packed, bu_p, bi_p)


# speedup vs baseline: 13.1206x; 1.0104x over previous
import functools

import jax
import jax.numpy as jnp
from jax import lax
from jax.experimental import pallas as pl
from jax.experimental.pallas import tpu as pltpu
from jax.experimental.pallas import tpu_sc as plsc

N_USERS = 1000000
N_ITEMS = 100000
K = 32
B = 16384
MU = 3.5

_INFO = plsc.get_sparse_core_info()
NC, NS, L = _INFO.num_cores, _INFO.num_subcores, _INFO.num_lanes
NW = NC * NS
BPW = B // NW
CB = 128
NCHUNK = BPW // CB
CGROUPS = CB // L


def _pack_body(pt_ref, out_ref):
    x = pt_ref[...]
    c = x.shape[1]
    r = c // 4
    y = x.reshape(K, 4, r).transpose(1, 0, 2).reshape(4 * K, r)
    out_ref[...] = y.T


def _pack(table_t, chunk):
    n = table_t.shape[1]
    grid = (n + chunk - 1) // chunk
    return pl.pallas_call(
        _pack_body,
        grid=(grid,),
        in_specs=[pl.BlockSpec((K, chunk), lambda c: (0, c))],
        out_specs=pl.BlockSpec((chunk // 4, 4 * K), lambda c: (c, 0)),
        out_shape=jax.ShapeDtypeStruct((grid * chunk // 4, 4 * K),
                                       jnp.float32),
    )(table_t)


def _lfm_kernel(uidx_hbm, iidx_hbm, p_hbm, q_hbm, bu_hbm, bi_hbm, out_hbm,
                uidx_v, iidx_v, sup_v, p_v, q_v, bu_v, bi_v, o_v, sem):
    wid = lax.axis_index("s") * NC + lax.axis_index("c")
    base = wid * BPW

    pltpu.sync_copy(uidx_hbm.at[pl.ds(base, BPW)], uidx_v)
    pltpu.sync_copy(iidx_hbm.at[pl.ds(base, BPW)], iidx_v)

    lane = lax.iota(jnp.int32, L)

    def chunk(c, carry):
        c0 = c * CB

        def mkidx(j, carry):
            u16 = uidx_v[pl.ds(c0 + j * L, L)]
            i16 = iidx_v[pl.ds(c0 + j * L, L)]
            sup_v[0, pl.ds(j * L, L)] = ((u16 >> 16) << 14) + (u16 & 16383)
            sup_v[1, pl.ds(j * L, L)] = ((i16 >> 12) << 10) + (i16 & 1023)
            sup_v[2, pl.ds(j * L, L)] = u16 >> 7
            sup_v[3, pl.ds(j * L, L)] = i16 >> 7
            return carry

        lax.fori_loop(0, CB // L, mkidx, 0)

        cps = [
            pltpu.async_copy(p_hbm.at[sup_v.at[0]], p_v, sem),
            pltpu.async_copy(q_hbm.at[sup_v.at[1]], q_v, sem),
            pltpu.async_copy(bu_hbm.at[sup_v.at[2]], bu_v, sem),
            pltpu.async_copy(bi_hbm.at[sup_v.at[3]], bi_v, sem),
        ]
        for cp in cps:
            cp.wait()

        def group(g, carry):
            rows = g * L + lane
            u16 = uidx_v[pl.ds(c0 + g * L, L)]
            i16 = iidx_v[pl.ds(c0 + g * L, L)]
            acc = (MU + plsc.load_gather(bu_v, [rows, u16 & 127])
                   + plsc.load_gather(bi_v, [rows, i16 & 127]))
            usub = ((u16 >> 14) & 3) * K
            isub = ((i16 >> 10) & 3) * K
            for k in range(K):
                pk = plsc.load_gather(p_v, [rows, usub + k])
                qk = plsc.load_gather(q_v, [rows, isub + k])
                acc = acc + pk * qk
            o_v[pl.ds(c0 + g * L, L)] = acc
            return carry

        lax.fori_loop(0, CGROUPS, group, 0)
        return carry

    lax.fori_loop(0, NCHUNK, chunk, 0)
    pltpu.sync_copy(o_v, out_hbm.at[pl.ds(base, BPW)])


@jax.jit
def kernel(user_idx, item_idx, P, Q, b_u, b_i):
    mesh = plsc.VectorSubcoreMesh(core_axis_name="c", subcore_axis_name="s")
    run = functools.partial(
        pl.kernel,
        mesh=mesh,
        out_type=jax.ShapeDtypeStruct((B,), jnp.float32),
        scratch_types=[
            pltpu.VMEM((BPW,), jnp.int32),
            pltpu.VMEM((BPW,), jnp.int32),
            pltpu.VMEM((4, CB), jnp.int32),
            pltpu.VMEM((CB, 128), jnp.float32),
            pltpu.VMEM((CB, 128), jnp.float32),
            pltpu.VMEM((CB, 128), jnp.float32),
            pltpu.VMEM((CB, 128), jnp.float32),
            pltpu.VMEM((BPW,), jnp.float32),
            pltpu.SemaphoreType.DMA,
        ],
        compiler_params=pltpu.CompilerParams(
            needs_layout_passes=False, use_tc_tiling_on_sc=True),
    )(_lfm_kernel)
    p_packed = _pack(P.T, 65536)
    q_packed = _pack(Q.T, 4096)
    bu_p = jnp.pad(b_u.T, ((0, 0), (0, 64))).reshape(-1, 128)
    bi_p = jnp.pad(b_i.T, ((0, 0), (0, 96))).reshape(-1, 128)
    return run(user_idx, item_idx, p_packed, q_packed, bu_p, bi_p)


# speedup vs baseline: 13.7139x; 1.0452x over previous
import functools

import jax
import jax.numpy as jnp
from jax import lax
from jax.experimental import pallas as pl
from jax.experimental.pallas import tpu as pltpu
from jax.experimental.pallas import tpu_sc as plsc

N_USERS = 1000000
N_ITEMS = 100000
K = 32
B = 16384
MU = 3.5

_INFO = plsc.get_sparse_core_info()
NC, NS, L = _INFO.num_cores, _INFO.num_subcores, _INFO.num_lanes
NW = NC * NS
BPW = B // NW
CB = 64
NCHUNK = BPW // CB
CGROUPS = CB // L


def _pack_body(pt_ref, out_ref):
    x = pt_ref[...]
    c = x.shape[1]
    r = c // 4
    y = x.reshape(K, 4, r).transpose(1, 0, 2).reshape(4 * K, r)
    out_ref[...] = y.T


def _pack(table_t, chunk):
    n = table_t.shape[1]
    grid = (n + chunk - 1) // chunk
    return pl.pallas_call(
        _pack_body,
        grid=(grid,),
        in_specs=[pl.BlockSpec((K, chunk), lambda c: (0, c))],
        out_specs=pl.BlockSpec((chunk // 4, 4 * K), lambda c: (c, 0)),
        out_shape=jax.ShapeDtypeStruct((grid * chunk // 4, 4 * K),
                                       jnp.float32),
    )(table_t)


def _lfm_kernel(uidx_hbm, iidx_hbm, p_hbm, q_hbm, bu_hbm, bi_hbm, out_hbm,
                uidx_v, iidx_v, sup_v, p_v, q_v, bu_v, bi_v, o_v, sem0, sem1):
    wid = lax.axis_index("s") * NC + lax.axis_index("c")
    base = wid * BPW

    pltpu.sync_copy(uidx_hbm.at[pl.ds(base, BPW)], uidx_v)
    pltpu.sync_copy(iidx_hbm.at[pl.ds(base, BPW)], iidx_v)

    lane = lax.iota(jnp.int32, L)

    def fire(c, s):
        c0 = c * CB
        for j in range(CB // L):
            u16 = uidx_v[pl.ds(c0 + j * L, L)]
            i16 = iidx_v[pl.ds(c0 + j * L, L)]
            sup_v[s, 0, pl.ds(j * L, L)] = ((u16 >> 16) << 14) + (u16 & 16383)
            sup_v[s, 1, pl.ds(j * L, L)] = ((i16 >> 12) << 10) + (i16 & 1023)
            sup_v[s, 2, pl.ds(j * L, L)] = u16 >> 7
            sup_v[s, 3, pl.ds(j * L, L)] = i16 >> 7
        sem = sem0 if s == 0 else sem1
        return [
            pltpu.async_copy(p_hbm.at[sup_v.at[s, 0]], p_v.at[s], sem),
            pltpu.async_copy(q_hbm.at[sup_v.at[s, 1]], q_v.at[s], sem),
            pltpu.async_copy(bu_hbm.at[sup_v.at[s, 2]], bu_v.at[s], sem),
            pltpu.async_copy(bi_hbm.at[sup_v.at[s, 3]], bi_v.at[s], sem),
        ]

    def compute(c, s):
        c0 = c * CB

        def group(g, carry):
            rows = g * L + lane
            u16 = uidx_v[pl.ds(c0 + g * L, L)]
            i16 = iidx_v[pl.ds(c0 + g * L, L)]
            acc = (MU + plsc.load_gather(bu_v, [jnp.full((L,), s, jnp.int32),
                                                rows, u16 & 127])
                   + plsc.load_gather(bi_v, [jnp.full((L,), s, jnp.int32),
                                             rows, i16 & 127]))
            usub = ((u16 >> 14) & 3) * K
            isub = ((i16 >> 10) & 3) * K
            ss = jnp.full((L,), s, jnp.int32)
            for k in range(K):
                pk = plsc.load_gather(p_v, [ss, rows, usub + k])
                qk = plsc.load_gather(q_v, [ss, rows, isub + k])
                acc = acc + pk * qk
            o_v[pl.ds(c0 + g * L, L)] = acc
            return carry

        lax.fori_loop(0, CGROUPS, group, 0)

    pending = fire(0, 0)
    for c in range(NCHUNK):
        s = c & 1
        if c + 1 < NCHUNK:
            nxt = fire(c + 1, 1 - s)
        else:
            nxt = None
        for cp in pending:
            cp.wait()
        compute(c, s)
        pending = nxt

    pltpu.sync_copy(o_v, out_hbm.at[pl.ds(base, BPW)])


@jax.jit
def kernel(user_idx, item_idx, P, Q, b_u, b_i):
    mesh = plsc.VectorSubcoreMesh(core_axis_name="c", subcore_axis_name="s")
    run = functools.partial(
        pl.kernel,
        mesh=mesh,
        out_type=jax.ShapeDtypeStruct((B,), jnp.float32),
        scratch_types=[
            pltpu.VMEM((BPW,), jnp.int32),
            pltpu.VMEM((BPW,), jnp.int32),
            pltpu.VMEM((2, 4, CB), jnp.int32),
            pltpu.VMEM((2, CB, 128), jnp.float32),
            pltpu.VMEM((2, CB, 128), jnp.float32),
            pltpu.VMEM((2, CB, 128), jnp.float32),
            pltpu.VMEM((2, CB, 128), jnp.float32),
            pltpu.VMEM((BPW,), jnp.float32),
            pltpu.SemaphoreType.DMA,
            pltpu.SemaphoreType.DMA,
        ],
        compiler_params=pltpu.CompilerParams(
            needs_layout_passes=False, use_tc_tiling_on_sc=True),
    )(_lfm_kernel)
    p_packed = _pack(P.T, 65536)
    q_packed = _pack(Q.T, 4096)
    bu_p = jnp.pad(b_u.T, ((0, 0), (0, 64))).reshape(-1, 128)
    bi_p = jnp.pad(b_i.T, ((0, 0), (0, 96))).reshape(-1, 128)
    return run(user_idx, item_idx, p_packed, q_packed, bu_p, bi_p)
